# baseline (speedup 1.0000x reference)
import jax
import jax.numpy as jnp
from jax.experimental import pallas as pl
from jax.experimental.pallas import tpu as pltpu

B = 1024
K = 100000
KB = 4096
NKB = (K + KB - 1) // KB


def _body(s_ref, v_ref, o_ref):
    k = pl.program_id(0)

    @pl.when(k == 0)
    def _():
        o_ref[...] = jnp.zeros_like(o_ref)

    kbase = k * KB
    col = kbase + jax.lax.broadcasted_iota(jnp.int32, (KB, 1), 0)
    vmask = col < K
    v = jnp.where(vmask, v_ref[...], 0.0)
    smask = (kbase + jax.lax.broadcasted_iota(jnp.int32, (1, KB), 1)) < K
    s = jnp.where(smask, s_ref[...], 0.0)
    o_ref[...] += jnp.dot(s, v, preferred_element_type=jnp.float32)


def kernel(state, values):
    return pl.pallas_call(
        _body,
        grid=(NKB,),
        in_specs=[
            pl.BlockSpec((B, KB), lambda k: (0, k)),
            pl.BlockSpec((KB, 1), lambda k: (k, 0)),
        ],
        out_specs=pl.BlockSpec((B, 1), lambda k: (0, 0)),
        out_shape=jax.ShapeDtypeStruct((B, 1), jnp.float32),
        compiler_params=pltpu.CompilerParams(
            dimension_semantics=("arbitrary",),
        ),
    )(state, values)


# speedup vs baseline: 1.0837x; 1.0837x over previous
import jax
import jax.numpy as jnp
from jax.experimental import pallas as pl
from jax.experimental.pallas import tpu as pltpu

B = 1024
K = 100000
KB = 4096
NFULL = K // KB
REM = K - NFULL * KB
NKB = NFULL + (1 if REM else 0)
LANES = 128
NSL = KB // LANES


def _body(s_ref, v_ref, o_ref, acc_ref):
    k = pl.program_id(0)

    @pl.when(k == 0)
    def _():
        acc_ref[...] = jnp.zeros_like(acc_ref)

    v = v_ref[...]
    s = s_ref[...]
    if REM:
        lane = jax.lax.broadcasted_iota(jnp.int32, (1, KB), 1)

        @pl.when(k == NKB - 1)
        def _():
            acc = acc_ref[...]
            sm = jnp.where(lane < REM, s, 0.0)
            vm = jnp.where(lane < REM, v, 0.0)
            for i in range(NSL):
                sl = slice(i * LANES, (i + 1) * LANES)
                acc += sm[:, sl] * vm[:, sl]
            acc_ref[...] = acc

        @pl.when(k < NKB - 1)
        def _():
            acc = acc_ref[...]
            for i in range(NSL):
                sl = slice(i * LANES, (i + 1) * LANES)
                acc += s[:, sl] * v[:, sl]
            acc_ref[...] = acc
    else:
        acc = acc_ref[...]
        for i in range(NSL):
            sl = slice(i * LANES, (i + 1) * LANES)
            acc += s[:, sl] * v[:, sl]
        acc_ref[...] = acc

    @pl.when(k == NKB - 1)
    def _():
        o_ref[...] = jnp.sum(acc_ref[...], axis=1, keepdims=True)


def kernel(state, values):
    values_row = values.reshape(1, K)
    return pl.pallas_call(
        _body,
        grid=(NKB,),
        in_specs=[
            pl.BlockSpec((B, KB), lambda k: (0, k)),
            pl.BlockSpec((1, KB), lambda k: (0, k)),
        ],
        out_specs=pl.BlockSpec((B, 1), lambda k: (0, 0)),
        out_shape=jax.ShapeDtypeStruct((B, 1), jnp.float32),
        scratch_shapes=[pltpu.VMEM((B, LANES), jnp.float32)],
        compiler_params=pltpu.CompilerParams(
            dimension_semantics=("arbitrary",),
        ),
    )(state, values_row)
